# baseline (speedup 1.0000x reference)
import functools
import math

import jax
import jax.numpy as jnp
from jax import lax
from jax.experimental import pallas as pl
from jax.experimental.pallas import tpu as pltpu
from jax.experimental.pallas import tpu_sc as plsc

_B, _N, _T, _E = 4, 1024, 12, 128
_H, _Dh, _P = 4, 32, 128
_K, _L = 16, 4
_BN = _B * _N
_ROWS = _BN * _K
_EPS = float(jnp.finfo(jnp.float32).eps)


def _proj_body(h_ref, wq_ref, bq_ref, wk_ref, bk_ref, wv_ref, bv_ref,
               q_ref, kv_ref):
    hlast = h_ref[:, _L - 1, :]
    hmean = (h_ref[:, 0, :] + h_ref[:, 1, :]
             + h_ref[:, 2, :] + h_ref[:, 3, :]) * (1.0 / _L)
    q_ref[...] = (jnp.dot(hlast, wq_ref[...],
                          preferred_element_type=jnp.float32) + bq_ref[...])
    k_f = (jnp.dot(hmean, wk_ref[...],
                   preferred_element_type=jnp.float32) + bk_ref[...])
    v_f = (jnp.dot(hmean, wv_ref[...],
                   preferred_element_type=jnp.float32) + bv_ref[...])
    kb = lax.bitcast_convert_type(
        k_f.astype(jnp.bfloat16).astype(jnp.float32), jnp.uint32)
    vb = lax.bitcast_convert_type(
        v_f.astype(jnp.bfloat16).astype(jnp.float32), jnp.uint32)
    kv_ref[...] = kb | (vb >> 16)


def _proj(hid_r, Wq, bq, Wk, bk, Wv, bv, interpret=False):
    blk = 512
    grid = _BN // blk
    w_spec = pl.BlockSpec((_E, _P), lambda i: (0, 0))
    b_spec = pl.BlockSpec((1, _P), lambda i: (0, 0))
    return pl.pallas_call(
        _proj_body,
        grid=(grid,),
        in_specs=[
            pl.BlockSpec((blk, _L, _E), lambda i: (i, 0, 0)),
            w_spec, b_spec, w_spec, b_spec, w_spec, b_spec,
        ],
        out_specs=[
            pl.BlockSpec((blk, _P), lambda i: (i, 0)),
            pl.BlockSpec((blk, _P), lambda i: (i, 0)),
        ],
        out_shape=[
            jax.ShapeDtypeStruct((_BN, _P), jnp.float32),
            jax.ShapeDtypeStruct((_BN, _P), jnp.uint32),
        ],
        interpret=interpret,
    )(hid_r, Wq, bq, Wk, bk, Wv, bv)


def _knn_body(s, xy_ref, xyt_ref, tau_ref, logw_ref, idx_ref, work_ref):
    blk = logw_ref.shape[0]
    pid = s
    xb = xy_ref[:, 0:1]
    yb = xy_ref[:, 1:2]
    xr = xyt_ref[0:1, :]
    yr = xyt_ref[1:2, :]
    x2b = xb * xb + yb * yb
    x2r = xr * xr + yr * yr
    xbb = xb.astype(jnp.bfloat16).astype(jnp.float32)
    ybb = yb.astype(jnp.bfloat16).astype(jnp.float32)
    xrb = xr.astype(jnp.bfloat16).astype(jnp.float32)
    yrb = yr.astype(jnp.bfloat16).astype(jnp.float32)
    g = xbb * xrb + ybb * yrb
    d2 = jnp.clip((x2b + x2r) - 2.0 * g, 0.0, None)
    colid = lax.broadcasted_iota(jnp.int32, (blk, _N), 1)
    rowid = lax.broadcasted_iota(jnp.int32, (blk, _N), 0) + pid * blk
    work_ref[...] = jnp.where(rowid == colid, jnp.inf, d2)

    lane16 = lax.broadcasted_iota(jnp.int32, (blk, _K), 1)

    def step(t, carry):
        kd, ki = carry
        w = work_ref[...]
        m = jnp.min(w, axis=1, keepdims=True)
        sel = jnp.where(w == m, colid, _N)
        idx = jnp.min(sel, axis=1, keepdims=True)
        kd = jnp.where(lane16 == t, m, kd)
        ki = jnp.where(lane16 == t, idx, ki)
        work_ref[...] = jnp.where(colid == idx, jnp.inf, w)
        return kd, ki

    kd0 = jnp.zeros((blk, _K), jnp.float32)
    ki0 = jnp.zeros((blk, _K), jnp.int32)
    kd, ki = lax.fori_loop(0, _K, step, (kd0, ki0))
    kd = jnp.sqrt(kd + 1e-09)

    local_scale = jnp.maximum(jnp.sum(kd, axis=1, keepdims=True) * (1.0 / _K),
                              _EPS)
    tau_safe = jnp.maximum(tau_ref[0, 0], 0.0001)
    logw_ref[...] = jnp.clip(-(kd / local_scale) / tau_safe, -10.0, 0.0)
    for b in range(_B):
        idx_ref[b] = ki + b * _N


def _knn(xy, xy_t, tau_b, s, interpret=False):
    blk = _N // 4
    return pl.pallas_call(
        functools.partial(_knn_body, s),
        grid=(1,),
        in_specs=[
            pl.BlockSpec((blk, 2), lambda i: (s, 0)),
            pl.BlockSpec((2, _N), lambda i: (0, 0)),
            pl.BlockSpec((1, 128), lambda i: (0, 0)),
        ],
        out_specs=[
            pl.BlockSpec((blk, _K), lambda i: (0, 0)),
            pl.BlockSpec((_B, blk, _K), lambda i: (0, 0, 0)),
        ],
        out_shape=[
            jax.ShapeDtypeStruct((blk, _K), jnp.float32),
            jax.ShapeDtypeStruct((_B, blk, _K), jnp.int32),
        ],
        scratch_shapes=[pltpu.VMEM((blk, _N), jnp.float32)],
        interpret=interpret,
    )(xy, xy_t, tau_b)


def _sc_gather(table, idx2):
    n_workers = 32
    rows = idx2.shape[0] * 128
    per_w = rows // n_workers
    ch = 128
    n_ch = per_w // ch

    mesh = plsc.VectorSubcoreMesh(core_axis_name="c", subcore_axis_name="s")

    @functools.partial(
        pl.kernel,
        mesh=mesh,
        out_type=jax.ShapeDtypeStruct((rows, _P), jnp.uint32),
        scratch_types=[
            pltpu.VMEM((n_ch, ch), jnp.int32),
            pltpu.VMEM((ch, _P), jnp.uint32),
            pltpu.VMEM((ch, _P), jnp.uint32),
            pltpu.VMEM((ch, _P), jnp.uint32),
            pltpu.SemaphoreType.DMA,
            pltpu.SemaphoreType.DMA,
            pltpu.SemaphoreType.DMA,
            pltpu.SemaphoreType.DMA,
            pltpu.SemaphoreType.DMA,
            pltpu.SemaphoreType.DMA,
        ],
    )
    def gk(table_hbm, idx_hbm, out_hbm, idx_v,
           buf0, buf1, buf2, gs0, gs1, gs2, ws0, ws1, ws2):
        wid = lax.axis_index("s") * 2 + lax.axis_index("c")
        base = wid * per_w
        pltpu.sync_copy(idx_hbm.at[pl.ds(wid * n_ch, n_ch)], idx_v)
        bufs = (buf0, buf1, buf2)
        gsems = (gs0, gs1, gs2)
        wsems = (ws0, ws1, ws2)

        def start(c):
            return pltpu.async_copy(
                table_hbm.at[idx_v.at[c]], bufs[c % 3], gsems[c % 3])

        gcp = [None, None, None]
        wcp = [None, None, None]
        gcp[0] = start(0)
        if n_ch > 1:
            gcp[1] = start(1)
        for c in range(n_ch):
            gcp[c % 3].wait()
            if c + 2 < n_ch:
                if wcp[(c + 2) % 3] is not None:
                    wcp[(c + 2) % 3].wait()
                gcp[(c + 2) % 3] = start(c + 2)
            wcp[c % 3] = pltpu.async_copy(
                bufs[c % 3], out_hbm.at[pl.ds(base + c * ch, ch)],
                wsems[c % 3])
        for c in range(max(0, n_ch - 3), n_ch):
            wcp[c % 3].wait()

    return gk(table, idx2)


def _attn_body(q_ref, kvg_ref, logw_ref, wo_ref, bo_ref, out_ref):
    blk = q_ref.shape[0]
    q = q_ref[...]
    kvm = kvg_ref[...]
    kvk = lax.bitcast_convert_type(kvm & jnp.uint32(0xFFFF0000), jnp.float32)
    kvv = lax.bitcast_convert_type(kvm << 16, jnp.float32)
    scale = 1.0 / math.sqrt(_Dh)
    s_r = lax.broadcasted_iota(jnp.int32, (_P, _H), 0) // _Dh
    s_c = lax.broadcasted_iota(jnp.int32, (_P, _H), 1)
    S = (s_r == s_c).astype(jnp.bfloat16)
    t_r = lax.broadcasted_iota(jnp.int32, (_H, _P), 0)
    t_c = lax.broadcasted_iota(jnp.int32, (_H, _P), 1) // _Dh
    St = (t_r == t_c).astype(jnp.bfloat16)

    def _dot1(x, s):
        return jnp.dot(x.astype(jnp.bfloat16), s,
                       preferred_element_type=jnp.float32)

    qm = jnp.reshape(jnp.broadcast_to(q[:, None, :], (blk, _K, _P)),
                     (blk * _K, _P))
    prod = qm * kvk
    logits_m = _dot1(prod, S)
    lg = (jnp.reshape(logits_m, (blk, _K, _H)) * scale
          + logw_ref[...][:, :, None])
    mx = jnp.max(lg, axis=1, keepdims=True)
    e = jnp.exp(lg - mx)
    a = e / jnp.sum(e, axis=1, keepdims=True)
    aw = _dot1(jnp.reshape(a, (blk * _K, _H)), St)
    ctx = jnp.sum(jnp.reshape(aw * kvv, (blk, _K, _P)), axis=1)
    out_ref[...] = (jnp.dot(ctx, wo_ref[...],
                            preferred_element_type=jnp.float32) + bo_ref[...])


def _attn(q, kvg, logw, woP, bo, s, interpret=False):
    blk = _N // 4
    per_batch = _N // blk
    return pl.pallas_call(
        _attn_body,
        grid=(_B,),
        in_specs=[
            pl.BlockSpec((blk, _P), lambda i: (i * per_batch + s, 0)),
            pl.BlockSpec((blk * _K, _P), lambda i: (i, 0)),
            pl.BlockSpec((blk, _K), lambda i: (0, 0)),
            pl.BlockSpec((_P, _E), lambda i: (0, 0)),
            pl.BlockSpec((1, _E), lambda i: (0, 0)),
        ],
        out_specs=pl.BlockSpec((blk, _E), lambda i: (i, 0)),
        out_shape=jax.ShapeDtypeStruct((_B * blk, _E), jnp.float32),
        interpret=interpret,
    )(q, kvg, logw, woP, bo)


def kernel(hidden, positions_xy, Wq, bq, Wk, bk, Wv, bv, Wo, bo, tau):
    hid_r = hidden.reshape(_BN, _T, _E)[:, _T - _L:, :]
    q, kv = _proj(hid_r, Wq, bq.reshape(1, _P), Wk, bk.reshape(1, _P),
                  Wv, bv.reshape(1, _P))
    xy_t = positions_xy.T
    tau_b = jnp.broadcast_to(jnp.reshape(tau, (1, 1)), (1, 128))
    woP = Wo.reshape(_Dh, _H, _E).transpose(1, 0, 2).reshape(_P, _E)
    parts = []
    for s in range(4):
        logw_s, idxf_s = _knn(positions_xy, xy_t, tau_b, s)
        idx2_s = idxf_s.reshape(_ROWS // 4 // 128, 128)
        kvg_s = _sc_gather(kv, idx2_s)
        out_s = _attn(q, kvg_s, logw_s, woP, bo.reshape(1, _E), s)
        parts.append(out_s.reshape(_B, _N // 4, _E))
    return jnp.concatenate(parts, axis=1)


# speedup vs baseline: 1.1596x; 1.1596x over previous
import functools
import math

import jax
import jax.numpy as jnp
from jax import lax
from jax.experimental import pallas as pl
from jax.experimental.pallas import tpu as pltpu
from jax.experimental.pallas import tpu_sc as plsc

_B, _N, _T, _E = 4, 1024, 12, 128
_H, _Dh, _P = 4, 32, 128
_K, _L = 16, 4
_BN = _B * _N
_ROWS = _BN * _K
_EPS = float(jnp.finfo(jnp.float32).eps)


def _proj_body(h_ref, wq_ref, bq_ref, wk_ref, bk_ref, wv_ref, bv_ref,
               q_ref, kv_ref):
    hlast = h_ref[:, _L - 1, :]
    hmean = (h_ref[:, 0, :] + h_ref[:, 1, :]
             + h_ref[:, 2, :] + h_ref[:, 3, :]) * (1.0 / _L)
    q_ref[...] = (jnp.dot(hlast, wq_ref[...],
                          preferred_element_type=jnp.float32) + bq_ref[...])
    k_f = (jnp.dot(hmean, wk_ref[...],
                   preferred_element_type=jnp.float32) + bk_ref[...])
    v_f = (jnp.dot(hmean, wv_ref[...],
                   preferred_element_type=jnp.float32) + bv_ref[...])
    kb = lax.bitcast_convert_type(
        k_f.astype(jnp.bfloat16).astype(jnp.float32), jnp.uint32)
    vb = lax.bitcast_convert_type(
        v_f.astype(jnp.bfloat16).astype(jnp.float32), jnp.uint32)
    kv_ref[...] = kb | (vb >> 16)


def _proj(hid_r, Wq, bq, Wk, bk, Wv, bv, interpret=False):
    blk = 512
    grid = _BN // blk
    w_spec = pl.BlockSpec((_E, _P), lambda i: (0, 0))
    b_spec = pl.BlockSpec((1, _P), lambda i: (0, 0))
    return pl.pallas_call(
        _proj_body,
        grid=(grid,),
        in_specs=[
            pl.BlockSpec((blk, _L, _E), lambda i: (i, 0, 0)),
            w_spec, b_spec, w_spec, b_spec, w_spec, b_spec,
        ],
        out_specs=[
            pl.BlockSpec((blk, _P), lambda i: (i, 0)),
            pl.BlockSpec((blk, _P), lambda i: (i, 0)),
        ],
        out_shape=[
            jax.ShapeDtypeStruct((_BN, _P), jnp.float32),
            jax.ShapeDtypeStruct((_BN, _P), jnp.uint32),
        ],
        interpret=interpret,
    )(hid_r, Wq, bq, Wk, bk, Wv, bv)


def _knn_body(s, xy_ref, xyt_ref, tau_ref, logw_ref, idx_ref, work_ref):
    blk = logw_ref.shape[0]
    pid = s
    xb = xy_ref[:, 0:1]
    yb = xy_ref[:, 1:2]
    xr = xyt_ref[0:1, :]
    yr = xyt_ref[1:2, :]
    x2b = xb * xb + yb * yb
    x2r = xr * xr + yr * yr
    xbb = xb.astype(jnp.bfloat16).astype(jnp.float32)
    ybb = yb.astype(jnp.bfloat16).astype(jnp.float32)
    xrb = xr.astype(jnp.bfloat16).astype(jnp.float32)
    yrb = yr.astype(jnp.bfloat16).astype(jnp.float32)
    g = xbb * xrb + ybb * yrb
    d2 = jnp.clip((x2b + x2r) - 2.0 * g, 0.0, None)
    colid = lax.broadcasted_iota(jnp.int32, (blk, _N), 1)
    rowid = lax.broadcasted_iota(jnp.int32, (blk, _N), 0) + pid * blk
    work_ref[...] = jnp.where(rowid == colid, jnp.inf, d2)

    lane16 = lax.broadcasted_iota(jnp.int32, (blk, _K), 1)

    def step(t, carry):
        kd, ki = carry
        w = work_ref[...]
        m = jnp.min(w, axis=1, keepdims=True)
        sel = jnp.where(w == m, colid, _N)
        idx = jnp.min(sel, axis=1, keepdims=True)
        kd = jnp.where(lane16 == t, m, kd)
        ki = jnp.where(lane16 == t, idx, ki)
        work_ref[...] = jnp.where(colid == idx, jnp.inf, w)
        return kd, ki

    kd0 = jnp.zeros((blk, _K), jnp.float32)
    ki0 = jnp.zeros((blk, _K), jnp.int32)
    kd, ki = lax.fori_loop(0, _K, step, (kd0, ki0))
    kd = jnp.sqrt(kd + 1e-09)

    local_scale = jnp.maximum(jnp.sum(kd, axis=1, keepdims=True) * (1.0 / _K),
                              _EPS)
    tau_safe = jnp.maximum(tau_ref[0, 0], 0.0001)
    logw_ref[...] = jnp.clip(-(kd / local_scale) / tau_safe, -10.0, 0.0)
    for b in range(_B):
        idx_ref[b] = ki + b * _N


_NSPLIT = 2


def _knn(xy, xy_t, tau_b, s, interpret=False):
    blk = _N // _NSPLIT
    return pl.pallas_call(
        functools.partial(_knn_body, s),
        grid=(1,),
        in_specs=[
            pl.BlockSpec((blk, 2), lambda i: (s, 0)),
            pl.BlockSpec((2, _N), lambda i: (0, 0)),
            pl.BlockSpec((1, 128), lambda i: (0, 0)),
        ],
        out_specs=[
            pl.BlockSpec((blk, _K), lambda i: (0, 0)),
            pl.BlockSpec((_B, blk, _K), lambda i: (0, 0, 0)),
        ],
        out_shape=[
            jax.ShapeDtypeStruct((blk, _K), jnp.float32),
            jax.ShapeDtypeStruct((_B, blk, _K), jnp.int32),
        ],
        scratch_shapes=[pltpu.VMEM((blk, _N), jnp.float32)],
        interpret=interpret,
    )(xy, xy_t, tau_b)


def _sc_gather(table, idx2):
    n_workers = 32
    rows = idx2.shape[0] * 128
    per_w = rows // n_workers
    ch = 128
    n_ch = per_w // ch

    mesh = plsc.VectorSubcoreMesh(core_axis_name="c", subcore_axis_name="s")

    @functools.partial(
        pl.kernel,
        mesh=mesh,
        out_type=jax.ShapeDtypeStruct((rows, _P), jnp.uint32),
        scratch_types=[
            pltpu.VMEM((n_ch, ch), jnp.int32),
            pltpu.VMEM((ch, _P), jnp.uint32),
            pltpu.VMEM((ch, _P), jnp.uint32),
            pltpu.VMEM((ch, _P), jnp.uint32),
            pltpu.SemaphoreType.DMA,
            pltpu.SemaphoreType.DMA,
            pltpu.SemaphoreType.DMA,
            pltpu.SemaphoreType.DMA,
            pltpu.SemaphoreType.DMA,
            pltpu.SemaphoreType.DMA,
        ],
    )
    def gk(table_hbm, idx_hbm, out_hbm, idx_v,
           buf0, buf1, buf2, gs0, gs1, gs2, ws0, ws1, ws2):
        wid = lax.axis_index("s") * 2 + lax.axis_index("c")
        base = wid * per_w
        pltpu.sync_copy(idx_hbm.at[pl.ds(wid * n_ch, n_ch)], idx_v)
        bufs = (buf0, buf1, buf2)
        gsems = (gs0, gs1, gs2)
        wsems = (ws0, ws1, ws2)

        def start(c):
            return pltpu.async_copy(
                table_hbm.at[idx_v.at[c]], bufs[c % 3], gsems[c % 3])

        gcp = [None, None, None]
        wcp = [None, None, None]
        gcp[0] = start(0)
        if n_ch > 1:
            gcp[1] = start(1)
        for c in range(n_ch):
            gcp[c % 3].wait()
            if c + 2 < n_ch:
                if wcp[(c + 2) % 3] is not None:
                    wcp[(c + 2) % 3].wait()
                gcp[(c + 2) % 3] = start(c + 2)
            wcp[c % 3] = pltpu.async_copy(
                bufs[c % 3], out_hbm.at[pl.ds(base + c * ch, ch)],
                wsems[c % 3])
        for c in range(max(0, n_ch - 3), n_ch):
            wcp[c % 3].wait()

    return gk(table, idx2)


def _attn_body(q_ref, kvg_ref, logw_ref, wo_ref, bo_ref, out_ref):
    blk = q_ref.shape[0]
    q = q_ref[...]
    kvm = kvg_ref[...]
    kvk = lax.bitcast_convert_type(kvm & jnp.uint32(0xFFFF0000), jnp.float32)
    kvv = lax.bitcast_convert_type(kvm << 16, jnp.float32)
    scale = 1.0 / math.sqrt(_Dh)
    s_r = lax.broadcasted_iota(jnp.int32, (_P, _H), 0) // _Dh
    s_c = lax.broadcasted_iota(jnp.int32, (_P, _H), 1)
    S = (s_r == s_c).astype(jnp.bfloat16)
    t_r = lax.broadcasted_iota(jnp.int32, (_H, _P), 0)
    t_c = lax.broadcasted_iota(jnp.int32, (_H, _P), 1) // _Dh
    St = (t_r == t_c).astype(jnp.bfloat16)

    def _dot1(x, s):
        return jnp.dot(x.astype(jnp.bfloat16), s,
                       preferred_element_type=jnp.float32)

    qm = jnp.reshape(jnp.broadcast_to(q[:, None, :], (blk, _K, _P)),
                     (blk * _K, _P))
    prod = qm * kvk
    logits_m = _dot1(prod, S)
    lg = (jnp.reshape(logits_m, (blk, _K, _H)) * scale
          + logw_ref[...][:, :, None])
    mx = jnp.max(lg, axis=1, keepdims=True)
    e = jnp.exp(lg - mx)
    a = e / jnp.sum(e, axis=1, keepdims=True)
    aw = _dot1(jnp.reshape(a, (blk * _K, _H)), St)
    ctx = jnp.sum(jnp.reshape(aw * kvv, (blk, _K, _P)), axis=1)
    out_ref[...] = (jnp.dot(ctx, wo_ref[...],
                            preferred_element_type=jnp.float32) + bo_ref[...])


def _attn(q, kvg, logw, woP, bo, s, interpret=False):
    blk = _N // _NSPLIT
    per_batch = _N // blk
    return pl.pallas_call(
        _attn_body,
        grid=(_B,),
        in_specs=[
            pl.BlockSpec((blk, _P), lambda i: (i * per_batch + s, 0)),
            pl.BlockSpec((blk * _K, _P), lambda i: (i, 0)),
            pl.BlockSpec((blk, _K), lambda i: (0, 0)),
            pl.BlockSpec((_P, _E), lambda i: (0, 0)),
            pl.BlockSpec((1, _E), lambda i: (0, 0)),
        ],
        out_specs=pl.BlockSpec((blk, _E), lambda i: (i, 0)),
        out_shape=jax.ShapeDtypeStruct((_B * blk, _E), jnp.float32),
        interpret=interpret,
    )(q, kvg, logw, woP, bo)


def kernel(hidden, positions_xy, Wq, bq, Wk, bk, Wv, bv, Wo, bo, tau):
    hid_r = hidden.reshape(_BN, _T, _E)[:, _T - _L:, :]
    q, kv = _proj(hid_r, Wq, bq.reshape(1, _P), Wk, bk.reshape(1, _P),
                  Wv, bv.reshape(1, _P))
    xy_t = positions_xy.T
    tau_b = jnp.broadcast_to(jnp.reshape(tau, (1, 1)), (1, 128))
    woP = Wo.reshape(_Dh, _H, _E).transpose(1, 0, 2).reshape(_P, _E)
    parts = []
    for s in range(_NSPLIT):
        logw_s, idxf_s = _knn(positions_xy, xy_t, tau_b, s)
        idx2_s = idxf_s.reshape(_ROWS // _NSPLIT // 128, 128)
        kvg_s = _sc_gather(kv, idx2_s)
        out_s = _attn(q, kvg_s, logw_s, woP, bo.reshape(1, _E), s)
        parts.append(out_s.reshape(_B, _N // _NSPLIT, _E))
    return jnp.concatenate(parts, axis=1)
